# baseline (speedup 1.0000x reference)
import jax
import jax.numpy as jnp
from jax.experimental import pallas as pl
from jax.experimental.pallas import tpu as pltpu

MAX_NUM_TILES = 4
NUM_PATCHES = 1025
HIDDEN_SIZE = 1280
NUM_ASPECT = 9
BATCH = 8


def _body(perm_ref, ids_ref, hid_ref, gate_ref, emb_ref, tab_ref, out_ref):
    g = jnp.tanh(gate_ref[0])
    out_ref[0, 0] = (
        hid_ref[0, 0]
        + (1.0 - g) * emb_ref[...]
        + g * tab_ref[0, 0]
    )


def kernel(hidden_state, aspect_ratio_ids, gate, embedding, tile_embedding_weight):
    ids = aspect_ratio_ids.astype(jnp.int32)
    perm = jnp.argsort(ids)
    ids_sorted = jnp.take(ids, perm, axis=0)
    table = tile_embedding_weight.reshape(NUM_ASPECT, MAX_NUM_TILES, NUM_PATCHES, HIDDEN_SIZE)

    grid_spec = pltpu.PrefetchScalarGridSpec(
        num_scalar_prefetch=2,
        grid=(MAX_NUM_TILES, BATCH),
        in_specs=[
            pl.BlockSpec(
                (1, 1, NUM_PATCHES, HIDDEN_SIZE),
                lambda t, b, perm, ids: (perm[b], t, 0, 0),
            ),
            pl.BlockSpec(memory_space=pltpu.SMEM),
            pl.BlockSpec(
                (NUM_PATCHES, HIDDEN_SIZE),
                lambda t, b, perm, ids: (0, 0),
            ),
            pl.BlockSpec(
                (1, 1, NUM_PATCHES, HIDDEN_SIZE),
                lambda t, b, perm, ids: (ids[b], t, 0, 0),
            ),
        ],
        out_specs=pl.BlockSpec(
            (1, 1, NUM_PATCHES, HIDDEN_SIZE),
            lambda t, b, perm, ids: (perm[b], t, 0, 0),
        ),
    )

    return pl.pallas_call(
        _body,
        grid_spec=grid_spec,
        out_shape=jax.ShapeDtypeStruct(hidden_state.shape, hidden_state.dtype),
    )(perm, ids_sorted, hidden_state, gate, embedding, table)


# speedup vs baseline: 3.3708x; 3.3708x over previous
import jax
import jax.numpy as jnp
from jax.experimental import pallas as pl
from jax.experimental.pallas import tpu as pltpu

MAX_NUM_TILES = 4
NUM_PATCHES = 1025
HIDDEN_SIZE = 1280
NUM_ASPECT = 9
BATCH = 8

ROW = NUM_PATCHES * HIDDEN_SIZE
PBLK = 128
CBLK = PBLK * HIDDEN_SIZE
KBLKS = (NUM_PATCHES - 1) // PBLK


def _main_body(perm_ref, ids_ref, hid_ref, gate_ref, emb_ref, tab_ref, out_ref):
    g = jnp.tanh(gate_ref[0])
    idv = ids_ref[pl.program_id(2)]
    row = tab_ref[idv]
    tab = row.reshape(PBLK, HIDDEN_SIZE)
    out_ref[0, 0] = hid_ref[0, 0] + (1.0 - g) * emb_ref[...] + g * tab


def _runt_body(perm_ref, ids_ref, hid_ref, gate_ref, emb_ref,
               t0_ref, t1_ref, t2_ref, t3_ref, out_ref):
    g = jnp.tanh(gate_ref[0])
    idv = ids_ref[pl.program_id(0)]
    base = (1.0 - g) * emb_ref[...]
    for t, tref in enumerate((t0_ref, t1_ref, t2_ref, t3_ref)):
        out_ref[0, t, 0] = hid_ref[0, t, 0] + base[0] + g * tref[idv]


def kernel(hidden_state, aspect_ratio_ids, gate, embedding, tile_embedding_weight):
    ids = aspect_ratio_ids.astype(jnp.int32)
    perm = jnp.argsort(ids)
    ids_sorted = jnp.take(ids, perm, axis=0)
    table = tile_embedding_weight

    main = pl.pallas_call(
        _main_body,
        grid_spec=pltpu.PrefetchScalarGridSpec(
            num_scalar_prefetch=2,
            grid=(MAX_NUM_TILES, KBLKS, BATCH),
            in_specs=[
                pl.BlockSpec(
                    (1, 1, PBLK, HIDDEN_SIZE),
                    lambda t, k, b, perm, ids: (perm[b], t, k, 0),
                ),
                pl.BlockSpec(memory_space=pltpu.SMEM),
                pl.BlockSpec(
                    (PBLK, HIDDEN_SIZE),
                    lambda t, k, b, perm, ids: (k, 0),
                ),
                pl.BlockSpec(
                    (pl.Element(NUM_ASPECT), pl.Element(CBLK)),
                    lambda t, k, b, perm, ids: (0, pl.multiple_of(t * ROW + k * CBLK, 1280)),
                ),
            ],
            out_specs=pl.BlockSpec(
                (1, 1, PBLK, HIDDEN_SIZE),
                lambda t, k, b, perm, ids: (perm[b], t, k, 0),
            ),
        ),
        out_shape=jax.ShapeDtypeStruct(hidden_state.shape, hidden_state.dtype),
    )(perm, ids_sorted, hidden_state, gate, embedding, table)

    runt_col = (NUM_PATCHES - 1) * HIDDEN_SIZE
    runt = pl.pallas_call(
        _runt_body,
        grid_spec=pltpu.PrefetchScalarGridSpec(
            num_scalar_prefetch=2,
            grid=(BATCH,),
            in_specs=[
                pl.BlockSpec(
                    (1, MAX_NUM_TILES, 8, HIDDEN_SIZE),
                    lambda b, perm, ids: (perm[b], 0, (NUM_PATCHES - 1) // 8, 0),
                ),
                pl.BlockSpec(memory_space=pltpu.SMEM),
                pl.BlockSpec(
                    (8, HIDDEN_SIZE),
                    lambda b, perm, ids: ((NUM_PATCHES - 1) // 8, 0),
                ),
            ] + [
                pl.BlockSpec(
                    (pl.Element(NUM_ASPECT), pl.Element(HIDDEN_SIZE)),
                    (lambda t: lambda b, perm, ids: (0, pl.multiple_of(t * ROW + runt_col, 1280)))(t),
                )
                for t in range(MAX_NUM_TILES)
            ],
            out_specs=pl.BlockSpec(
                (1, MAX_NUM_TILES, 1, HIDDEN_SIZE),
                lambda b, perm, ids: (perm[b], 0, 0, 0),
            ),
        ),
        out_shape=jax.ShapeDtypeStruct(
            (BATCH, MAX_NUM_TILES, 1, HIDDEN_SIZE), hidden_state.dtype),
    )(perm, ids_sorted, hidden_state, gate, embedding, table, table, table, table)

    return jax.lax.dynamic_update_slice(main, runt, (0, 0, NUM_PATCHES - 1, 0))


# speedup vs baseline: 5.8750x; 1.7429x over previous
import jax
import jax.numpy as jnp
from jax.experimental import pallas as pl
from jax.experimental.pallas import tpu as pltpu

MAX_NUM_TILES = 4
NUM_PATCHES = 1025
HIDDEN_SIZE = 1280
NUM_ASPECT = 9
BATCH = 8

ROW = NUM_PATCHES * HIDDEN_SIZE
PBLK = 128
CBLK = PBLK * HIDDEN_SIZE
KBLKS = (NUM_PATCHES - 1) // PBLK


def _main_body(perm_ref, ids_ref, hid_ref, gate_ref, emb_ref, tab_ref, out_ref):
    g = jnp.tanh(gate_ref[0])
    idv = ids_ref[pl.program_id(2)]
    row = tab_ref[idv]
    tab = row.reshape(PBLK, HIDDEN_SIZE)
    out_ref[0, 0] = hid_ref[0, 0] + (1.0 - g) * emb_ref[...] + g * tab


def _runt_body(perm_ref, ids_ref, hid_ref, gate_ref, emb_ref,
               t0_ref, t1_ref, t2_ref, t3_ref, main_ref, out_ref):
    del main_ref
    g = jnp.tanh(gate_ref[0])
    idv = ids_ref[pl.program_id(0)]
    base = (1.0 - g) * emb_ref[...]
    for t, tref in enumerate((t0_ref, t1_ref, t2_ref, t3_ref)):
        out_ref[0, t, 0] = hid_ref[0, t, 0] + base[0] + g * tref[idv]


def kernel(hidden_state, aspect_ratio_ids, gate, embedding, tile_embedding_weight):
    ids = aspect_ratio_ids.astype(jnp.int32)
    perm = jnp.argsort(ids)
    ids_sorted = jnp.take(ids, perm, axis=0)
    table = tile_embedding_weight

    main = pl.pallas_call(
        _main_body,
        grid_spec=pltpu.PrefetchScalarGridSpec(
            num_scalar_prefetch=2,
            grid=(MAX_NUM_TILES, KBLKS, BATCH),
            in_specs=[
                pl.BlockSpec(
                    (1, 1, PBLK, HIDDEN_SIZE),
                    lambda t, k, b, perm, ids: (perm[b], t, k, 0),
                ),
                pl.BlockSpec(memory_space=pltpu.SMEM),
                pl.BlockSpec(
                    (PBLK, HIDDEN_SIZE),
                    lambda t, k, b, perm, ids: (k, 0),
                ),
                pl.BlockSpec(
                    (pl.Element(NUM_ASPECT), pl.Element(CBLK)),
                    lambda t, k, b, perm, ids: (0, pl.multiple_of(t * ROW + k * CBLK, 1280)),
                ),
            ],
            out_specs=pl.BlockSpec(
                (1, 1, PBLK, HIDDEN_SIZE),
                lambda t, k, b, perm, ids: (perm[b], t, k, 0),
            ),
        ),
        out_shape=jax.ShapeDtypeStruct(hidden_state.shape, hidden_state.dtype),
    )(perm, ids_sorted, hidden_state, gate, embedding, table)

    runt_col = (NUM_PATCHES - 1) * HIDDEN_SIZE
    runt = pl.pallas_call(
        _runt_body,
        grid_spec=pltpu.PrefetchScalarGridSpec(
            num_scalar_prefetch=2,
            grid=(BATCH,),
            in_specs=[
                pl.BlockSpec(
                    (1, MAX_NUM_TILES, 8, HIDDEN_SIZE),
                    lambda b, perm, ids: (perm[b], 0, (NUM_PATCHES - 1) // 8, 0),
                ),
                pl.BlockSpec(memory_space=pltpu.SMEM),
                pl.BlockSpec(
                    (8, HIDDEN_SIZE),
                    lambda b, perm, ids: ((NUM_PATCHES - 1) // 8, 0),
                ),
            ] + [
                pl.BlockSpec(
                    (pl.Element(NUM_ASPECT), pl.Element(HIDDEN_SIZE)),
                    (lambda t: lambda b, perm, ids: (0, pl.multiple_of(t * ROW + runt_col, 1280)))(t),
                )
                for t in range(MAX_NUM_TILES)
            ] + [
                pl.BlockSpec(memory_space=pl.ANY),
            ],
            out_specs=pl.BlockSpec(
                (1, MAX_NUM_TILES, 8, HIDDEN_SIZE),
                lambda b, perm, ids: (perm[b], 0, (NUM_PATCHES - 1) // 8, 0),
            ),
        ),
        out_shape=jax.ShapeDtypeStruct(hidden_state.shape, hidden_state.dtype),
        input_output_aliases={9: 0},
    )(perm, ids_sorted, hidden_state, gate, embedding, table, table, table, table,
      main)

    return runt


# speedup vs baseline: 5.9089x; 1.0058x over previous
import jax
import jax.numpy as jnp
from jax.experimental import pallas as pl
from jax.experimental.pallas import tpu as pltpu

MAX_NUM_TILES = 4
NUM_PATCHES = 1025
HIDDEN_SIZE = 1280
NUM_ASPECT = 9
BATCH = 8

ROW = NUM_PATCHES * HIDDEN_SIZE
PBLK = 128
CBLK = PBLK * HIDDEN_SIZE
KBLKS = (NUM_PATCHES - 1) // PBLK


def _main_body(perm_ref, ids_ref, hid_ref, gate_ref, emb_ref, tab_ref, out_ref):
    g = jnp.tanh(gate_ref[0])
    idv = ids_ref[pl.program_id(2)]
    row = tab_ref[idv]
    tab = row.reshape(PBLK, HIDDEN_SIZE)
    out_ref[0, 0] = hid_ref[0, 0] + (1.0 - g) * emb_ref[...] + g * tab


def _runt_body(perm_ref, ids_ref, hid_ref, gate_ref, emb_ref,
               t0_ref, t1_ref, t2_ref, t3_ref, main_ref, out_ref):
    del main_ref
    g = jnp.tanh(gate_ref[0])
    idv = ids_ref[pl.program_id(0)]
    base = (1.0 - g) * emb_ref[...]
    for t, tref in enumerate((t0_ref, t1_ref, t2_ref, t3_ref)):
        out_ref[0, t, 0] = hid_ref[0, t, 0] + base[0] + g * tref[idv]


def kernel(hidden_state, aspect_ratio_ids, gate, embedding, tile_embedding_weight):
    ids = aspect_ratio_ids.astype(jnp.int32)
    perm = jnp.argsort(ids)
    ids_sorted = jnp.take(ids, perm, axis=0)
    table = tile_embedding_weight

    main = pl.pallas_call(
        _main_body,
        grid_spec=pltpu.PrefetchScalarGridSpec(
            num_scalar_prefetch=2,
            grid=(KBLKS, MAX_NUM_TILES, BATCH),
            in_specs=[
                pl.BlockSpec(
                    (1, 1, PBLK, HIDDEN_SIZE),
                    lambda k, t, b, perm, ids: (perm[b], t, k, 0),
                ),
                pl.BlockSpec(memory_space=pltpu.SMEM),
                pl.BlockSpec(
                    (PBLK, HIDDEN_SIZE),
                    lambda k, t, b, perm, ids: (k, 0),
                    pipeline_mode=pl.Buffered(buffer_count=2),
                ),
                pl.BlockSpec(
                    (pl.Element(NUM_ASPECT), pl.Element(CBLK)),
                    lambda k, t, b, perm, ids: (0, pl.multiple_of(t * ROW + k * CBLK, 1280)),
                    pipeline_mode=pl.Buffered(buffer_count=2),
                ),
            ],
            out_specs=pl.BlockSpec(
                (1, 1, PBLK, HIDDEN_SIZE),
                lambda k, t, b, perm, ids: (perm[b], t, k, 0),
            ),
        ),
        out_shape=jax.ShapeDtypeStruct(hidden_state.shape, hidden_state.dtype),
    )(perm, ids_sorted, hidden_state, gate, embedding, table)

    runt_col = (NUM_PATCHES - 1) * HIDDEN_SIZE
    runt = pl.pallas_call(
        _runt_body,
        grid_spec=pltpu.PrefetchScalarGridSpec(
            num_scalar_prefetch=2,
            grid=(BATCH,),
            in_specs=[
                pl.BlockSpec(
                    (1, MAX_NUM_TILES, 8, HIDDEN_SIZE),
                    lambda b, perm, ids: (perm[b], 0, (NUM_PATCHES - 1) // 8, 0),
                ),
                pl.BlockSpec(memory_space=pltpu.SMEM),
                pl.BlockSpec(
                    (8, HIDDEN_SIZE),
                    lambda b, perm, ids: ((NUM_PATCHES - 1) // 8, 0),
                ),
            ] + [
                pl.BlockSpec(
                    (pl.Element(NUM_ASPECT), pl.Element(HIDDEN_SIZE)),
                    (lambda t: lambda b, perm, ids: (0, pl.multiple_of(t * ROW + runt_col, 1280)))(t),
                )
                for t in range(MAX_NUM_TILES)
            ] + [
                pl.BlockSpec(memory_space=pl.ANY),
            ],
            out_specs=pl.BlockSpec(
                (1, MAX_NUM_TILES, 8, HIDDEN_SIZE),
                lambda b, perm, ids: (perm[b], 0, (NUM_PATCHES - 1) // 8, 0),
            ),
        ),
        out_shape=jax.ShapeDtypeStruct(hidden_state.shape, hidden_state.dtype),
        input_output_aliases={9: 0},
    )(perm, ids_sorted, hidden_state, gate, embedding, table, table, table, table,
      main)

    return runt


# speedup vs baseline: 6.1307x; 1.0375x over previous
import jax
import jax.numpy as jnp
from jax.experimental import pallas as pl
from jax.experimental.pallas import tpu as pltpu

MAX_NUM_TILES = 4
NUM_PATCHES = 1025
HIDDEN_SIZE = 1280
NUM_ASPECT = 9
BATCH = 8

ROW = NUM_PATCHES * HIDDEN_SIZE
PBLK = 128
CBLK = PBLK * HIDDEN_SIZE
KBLKS = (NUM_PATCHES - 1) // PBLK


NWIN = KBLKS * MAX_NUM_TILES


def _win_copy(tab_hbm, buf_ref, sem_ref, w, slot):
    c0 = (w % MAX_NUM_TILES) * ROW + (w // MAX_NUM_TILES) * CBLK
    c0 = pl.multiple_of(c0, HIDDEN_SIZE)
    return pltpu.make_async_copy(
        tab_hbm.at[:, pl.ds(c0, CBLK)],
        buf_ref.at[slot],
        sem_ref.at[slot],
    )


def _main_body(perm_ref, ids_ref, hid_ref, gate_ref, emb_ref, tab_hbm,
               out_ref, buf_ref, sem_ref):
    k = pl.program_id(0)
    t = pl.program_id(1)
    b = pl.program_id(2)
    w = k * MAX_NUM_TILES + t
    slot = jax.lax.rem(w, 2)

    @pl.when((w == 0) & (b == 0))
    def _prologue():
        _win_copy(tab_hbm, buf_ref, sem_ref, w, slot).start()

    @pl.when((b == 0) & (w + 1 < NWIN))
    def _prefetch_next():
        _win_copy(tab_hbm, buf_ref, sem_ref, w + 1, jax.lax.rem(w + 1, 2)).start()

    @pl.when(b == 0)
    def _wait_current():
        _win_copy(tab_hbm, buf_ref, sem_ref, w, slot).wait()

    g = jnp.tanh(gate_ref[0])
    idv = ids_ref[b]
    row = buf_ref[slot, idv]
    tab = row.reshape(PBLK, HIDDEN_SIZE)
    out_ref[0, 0] = hid_ref[0, 0] + (1.0 - g) * emb_ref[...] + g * tab


def _runt_body(perm_ref, ids_ref, hid_ref, gate_ref, emb_ref,
               t0_ref, t1_ref, t2_ref, t3_ref, main_ref, out_ref):
    del main_ref
    g = jnp.tanh(gate_ref[0])
    idv = ids_ref[pl.program_id(0)]
    base = (1.0 - g) * emb_ref[...]
    for t, tref in enumerate((t0_ref, t1_ref, t2_ref, t3_ref)):
        out_ref[0, t, 0] = hid_ref[0, t, 0] + base[0] + g * tref[idv]


def kernel(hidden_state, aspect_ratio_ids, gate, embedding, tile_embedding_weight):
    ids = aspect_ratio_ids.astype(jnp.int32)
    perm = jnp.argsort(ids)
    ids_sorted = jnp.take(ids, perm, axis=0)
    table = tile_embedding_weight

    main = pl.pallas_call(
        _main_body,
        grid_spec=pltpu.PrefetchScalarGridSpec(
            num_scalar_prefetch=2,
            grid=(KBLKS, MAX_NUM_TILES, BATCH),
            in_specs=[
                pl.BlockSpec(
                    (1, 1, PBLK, HIDDEN_SIZE),
                    lambda k, t, b, perm, ids: (perm[b], t, k, 0),
                ),
                pl.BlockSpec(memory_space=pltpu.SMEM),
                pl.BlockSpec(
                    (PBLK, HIDDEN_SIZE),
                    lambda k, t, b, perm, ids: (k, 0),
                    pipeline_mode=pl.Buffered(buffer_count=2),
                ),
                pl.BlockSpec(memory_space=pl.ANY),
            ],
            out_specs=pl.BlockSpec(
                (1, 1, PBLK, HIDDEN_SIZE),
                lambda k, t, b, perm, ids: (perm[b], t, k, 0),
            ),
            scratch_shapes=[
                pltpu.VMEM((2, NUM_ASPECT, CBLK), jnp.float32),
                pltpu.SemaphoreType.DMA((2,)),
            ],
        ),
        out_shape=jax.ShapeDtypeStruct(hidden_state.shape, hidden_state.dtype),
    )(perm, ids_sorted, hidden_state, gate, embedding, table)

    runt_col = (NUM_PATCHES - 1) * HIDDEN_SIZE
    runt = pl.pallas_call(
        _runt_body,
        grid_spec=pltpu.PrefetchScalarGridSpec(
            num_scalar_prefetch=2,
            grid=(BATCH,),
            in_specs=[
                pl.BlockSpec(
                    (1, MAX_NUM_TILES, 8, HIDDEN_SIZE),
                    lambda b, perm, ids: (perm[b], 0, (NUM_PATCHES - 1) // 8, 0),
                ),
                pl.BlockSpec(memory_space=pltpu.SMEM),
                pl.BlockSpec(
                    (8, HIDDEN_SIZE),
                    lambda b, perm, ids: ((NUM_PATCHES - 1) // 8, 0),
                ),
            ] + [
                pl.BlockSpec(
                    (pl.Element(NUM_ASPECT), pl.Element(HIDDEN_SIZE)),
                    (lambda t: lambda b, perm, ids: (0, pl.multiple_of(t * ROW + runt_col, 1280)))(t),
                )
                for t in range(MAX_NUM_TILES)
            ] + [
                pl.BlockSpec(memory_space=pl.ANY),
            ],
            out_specs=pl.BlockSpec(
                (1, MAX_NUM_TILES, 8, HIDDEN_SIZE),
                lambda b, perm, ids: (perm[b], 0, (NUM_PATCHES - 1) // 8, 0),
            ),
        ),
        out_shape=jax.ShapeDtypeStruct(hidden_state.shape, hidden_state.dtype),
        input_output_aliases={9: 0},
    )(perm, ids_sorted, hidden_state, gate, embedding, table, table, table, table,
      main)

    return runt


# speedup vs baseline: 6.6240x; 1.0805x over previous
import jax
import jax.numpy as jnp
from jax.experimental import pallas as pl
from jax.experimental.pallas import tpu as pltpu

MAX_NUM_TILES = 4
NUM_PATCHES = 1025
HIDDEN_SIZE = 1280
NUM_ASPECT = 9
BATCH = 8

ROW = NUM_PATCHES * HIDDEN_SIZE
PBLK = 256
CBLK = PBLK * HIDDEN_SIZE
KBLKS = (NUM_PATCHES - 1) // PBLK


NWIN = KBLKS * MAX_NUM_TILES


def _win_copy(tab_hbm, buf_ref, sem_ref, w, slot):
    c0 = (w % MAX_NUM_TILES) * ROW + (w // MAX_NUM_TILES) * CBLK
    c0 = pl.multiple_of(c0, HIDDEN_SIZE)
    return pltpu.make_async_copy(
        tab_hbm.at[:, pl.ds(c0, CBLK)],
        buf_ref.at[slot],
        sem_ref.at[slot],
    )


def _main_body(perm_ref, ids_ref, hid_ref, gate_ref, emb_ref, tab_hbm,
               out_ref, buf_ref, sem_ref):
    k = pl.program_id(0)
    t = pl.program_id(1)
    b = pl.program_id(2)
    w = k * MAX_NUM_TILES + t
    slot = jax.lax.rem(w, 2)

    @pl.when((w == 0) & (b == 0))
    def _prologue():
        _win_copy(tab_hbm, buf_ref, sem_ref, w, slot).start()

    @pl.when((b == 0) & (w + 1 < NWIN))
    def _prefetch_next():
        _win_copy(tab_hbm, buf_ref, sem_ref, w + 1, jax.lax.rem(w + 1, 2)).start()

    @pl.when(b == 0)
    def _wait_current():
        _win_copy(tab_hbm, buf_ref, sem_ref, w, slot).wait()

    g = jnp.tanh(gate_ref[0])
    idv = ids_ref[b]
    row = buf_ref[slot, idv]
    tab = row.reshape(PBLK, HIDDEN_SIZE)
    out_ref[0, 0] = hid_ref[0, 0] + (1.0 - g) * emb_ref[...] + g * tab


def _runt_body(perm_ref, ids_ref, hid_ref, gate_ref, emb_ref,
               t0_ref, t1_ref, t2_ref, t3_ref, main_ref, out_ref):
    del main_ref
    g = jnp.tanh(gate_ref[0])
    idv = ids_ref[pl.program_id(0)]
    base = (1.0 - g) * emb_ref[...]
    for t, tref in enumerate((t0_ref, t1_ref, t2_ref, t3_ref)):
        out_ref[0, t, 0] = hid_ref[0, t, 0] + base[0] + g * tref[idv]


def kernel(hidden_state, aspect_ratio_ids, gate, embedding, tile_embedding_weight):
    ids = aspect_ratio_ids.astype(jnp.int32)
    perm = jnp.argsort(ids)
    ids_sorted = jnp.take(ids, perm, axis=0)
    table = tile_embedding_weight

    main = pl.pallas_call(
        _main_body,
        grid_spec=pltpu.PrefetchScalarGridSpec(
            num_scalar_prefetch=2,
            grid=(KBLKS, MAX_NUM_TILES, BATCH),
            in_specs=[
                pl.BlockSpec(
                    (1, 1, PBLK, HIDDEN_SIZE),
                    lambda k, t, b, perm, ids: (perm[b], t, k, 0),
                ),
                pl.BlockSpec(memory_space=pltpu.SMEM),
                pl.BlockSpec(
                    (PBLK, HIDDEN_SIZE),
                    lambda k, t, b, perm, ids: (k, 0),
                    pipeline_mode=pl.Buffered(buffer_count=2),
                ),
                pl.BlockSpec(memory_space=pl.ANY),
            ],
            out_specs=pl.BlockSpec(
                (1, 1, PBLK, HIDDEN_SIZE),
                lambda k, t, b, perm, ids: (perm[b], t, k, 0),
            ),
            scratch_shapes=[
                pltpu.VMEM((2, NUM_ASPECT, CBLK), jnp.float32),
                pltpu.SemaphoreType.DMA((2,)),
            ],
        ),
        out_shape=jax.ShapeDtypeStruct(hidden_state.shape, hidden_state.dtype),
    )(perm, ids_sorted, hidden_state, gate, embedding, table)

    runt_col = (NUM_PATCHES - 1) * HIDDEN_SIZE
    runt = pl.pallas_call(
        _runt_body,
        grid_spec=pltpu.PrefetchScalarGridSpec(
            num_scalar_prefetch=2,
            grid=(BATCH,),
            in_specs=[
                pl.BlockSpec(
                    (1, MAX_NUM_TILES, 8, HIDDEN_SIZE),
                    lambda b, perm, ids: (perm[b], 0, (NUM_PATCHES - 1) // 8, 0),
                ),
                pl.BlockSpec(memory_space=pltpu.SMEM),
                pl.BlockSpec(
                    (8, HIDDEN_SIZE),
                    lambda b, perm, ids: ((NUM_PATCHES - 1) // 8, 0),
                ),
            ] + [
                pl.BlockSpec(
                    (pl.Element(NUM_ASPECT), pl.Element(HIDDEN_SIZE)),
                    (lambda t: lambda b, perm, ids: (0, pl.multiple_of(t * ROW + runt_col, 1280)))(t),
                )
                for t in range(MAX_NUM_TILES)
            ] + [
                pl.BlockSpec(memory_space=pl.ANY),
            ],
            out_specs=pl.BlockSpec(
                (1, MAX_NUM_TILES, 8, HIDDEN_SIZE),
                lambda b, perm, ids: (perm[b], 0, (NUM_PATCHES - 1) // 8, 0),
            ),
        ),
        out_shape=jax.ShapeDtypeStruct(hidden_state.shape, hidden_state.dtype),
        input_output_aliases={9: 0},
    )(perm, ids_sorted, hidden_state, gate, embedding, table, table, table, table,
      main)

    return runt
